# Initial kernel scaffold; baseline (speedup 1.0000x reference)
import jax
import jax.numpy as jnp
from jax.experimental import pallas as pl


def kernel(inputs, embedding_matrix):
    raise NotImplementedError("write your pallas kernel here")


# speedup vs baseline: 1.4800x; 1.4800x over previous
import jax
import jax.numpy as jnp
from jax import lax
from jax.experimental import pallas as pl
from jax.experimental.pallas import tpu as pltpu
from jax.experimental.pallas import tpu_sc as plsc

VOCAB = 1000000
EMBED_DIM = 32
BATCH = 4096
HIST_LEN = 200

NC = 2
NS = 16
NW = NC * NS

ROWS_PER_STREAM = 128
TOTAL_ROWS = BATCH * HIST_LEN
ROWS_PER_W = TOTAL_ROWS // NW
CHUNKS = ROWS_PER_W // ROWS_PER_STREAM
GROUP = 8
GROUPS = CHUNKS // GROUP


def _make_gather():
    mesh = plsc.VectorSubcoreMesh(core_axis_name="c", subcore_axis_name="s")

    def body(idx_hbm, table_hbm, out_hbm, idx_v, rows_v, sem_g, sem_o):
        wid = lax.axis_index("s") * NC + lax.axis_index("c")
        pltpu.sync_copy(idx_hbm.at[wid], idx_v)

        def group(g, carry):
            base = g * GROUP
            descs = []
            for b in range(GROUP):
                descs.append(
                    pltpu.async_copy(
                        table_hbm.at[idx_v.at[base + b]],
                        rows_v.at[b],
                        sem_g,
                    )
                )
            for d in descs:
                d.wait()
            pltpu.async_copy(
                rows_v, out_hbm.at[wid, pl.ds(base, GROUP)], sem_o
            ).wait()
            return carry

        lax.fori_loop(0, GROUPS, group, 0)

    kern = pl.kernel(
        body,
        out_type=jax.ShapeDtypeStruct((NW, CHUNKS, ROWS_PER_STREAM, EMBED_DIM),
                                      jnp.float32),
        mesh=mesh,
        scratch_types=[
            pltpu.VMEM((CHUNKS, ROWS_PER_STREAM), jnp.int32),
            pltpu.VMEM((GROUP, ROWS_PER_STREAM, EMBED_DIM), jnp.float32),
            pltpu.SemaphoreType.DMA,
            pltpu.SemaphoreType.DMA,
        ],
        compiler_params=pltpu.CompilerParams(use_tc_tiling_on_sc=False),
    )
    return kern


_gather = _make_gather()


def kernel(inputs, embedding_matrix):
    idx = inputs.astype(jnp.int32).reshape(NW, CHUNKS, ROWS_PER_STREAM)
    out = _gather(idx, embedding_matrix)
    return out.reshape(BATCH, HIST_LEN, EMBED_DIM)


# speedup vs baseline: 1.4937x; 1.0093x over previous
import jax
import jax.numpy as jnp
from jax import lax
from jax.experimental import pallas as pl
from jax.experimental.pallas import tpu as pltpu
from jax.experimental.pallas import tpu_sc as plsc

VOCAB = 1000000
EMBED_DIM = 32
BATCH = 4096
HIST_LEN = 200

NC = 2
NS = 16
NW = NC * NS

ROWS_PER_STREAM = 128
TOTAL_ROWS = BATCH * HIST_LEN
ROWS_PER_W = TOTAL_ROWS // NW
CHUNKS = ROWS_PER_W // ROWS_PER_STREAM
GROUP = 8
GROUPS = CHUNKS // GROUP


def _make_gather():
    mesh = plsc.VectorSubcoreMesh(core_axis_name="c", subcore_axis_name="s")

    def body(idx_hbm, table_hbm, out_hbm, idx_v, rows_v, sem_g, sem_o):
        wid = lax.axis_index("s") * NC + lax.axis_index("c")
        pltpu.sync_copy(idx_hbm.at[wid], idx_v)

        def out_desc(g, slot):
            return pltpu.make_async_copy(
                rows_v.at[slot],
                out_hbm.at[wid, pl.ds(g * GROUP, GROUP)],
                sem_o,
            )

        def group(g, carry):
            slot = lax.rem(g, 2)
            base = g * GROUP

            @pl.when(g >= 2)
            def _():
                out_desc(g - 2, slot).wait()

            descs = []
            for b in range(GROUP):
                descs.append(
                    pltpu.async_copy(
                        table_hbm.at[idx_v.at[base + b]],
                        rows_v.at[slot, b],
                        sem_g,
                    )
                )
            for d in descs:
                d.wait()
            out_desc(g, slot).start()
            return carry

        lax.fori_loop(0, GROUPS, group, 0)
        for g in (GROUPS - 2, GROUPS - 1):
            out_desc(g, g % 2).wait()

    kern = pl.kernel(
        body,
        out_type=jax.ShapeDtypeStruct((NW, CHUNKS, ROWS_PER_STREAM, EMBED_DIM),
                                      jnp.float32),
        mesh=mesh,
        scratch_types=[
            pltpu.VMEM((CHUNKS, ROWS_PER_STREAM), jnp.int32),
            pltpu.VMEM((2, GROUP, ROWS_PER_STREAM, EMBED_DIM), jnp.float32),
            pltpu.SemaphoreType.DMA,
            pltpu.SemaphoreType.DMA,
        ],
        compiler_params=pltpu.CompilerParams(use_tc_tiling_on_sc=False),
    )
    return kern


_gather = _make_gather()


def kernel(inputs, embedding_matrix):
    idx = inputs.astype(jnp.int32).reshape(NW, CHUNKS, ROWS_PER_STREAM)
    out = _gather(idx, embedding_matrix)
    return out.reshape(BATCH, HIST_LEN, EMBED_DIM)
